# baseline (speedup 1.0000x reference)
import jax
import jax.numpy as jnp
from jax import lax
from jax.experimental import pallas as pl
from jax.experimental.pallas import tpu as pltpu
from jax.experimental.pallas import tpu_sc as plsc

NC = 2
NS = 16
NW = NC * NS

B = 16384
D = 32
DW = 128
PACK = DW // D
HIST = 50
BPW = B // NW
GCH = 128
NG = BPW // GCH

RB = 2048


def _sc_gather_body(uhi, ihi, uidx, iidx, eu, ei, ul, ub, ib,
                    ue_o, ie_o, ul_o, ub_o, ib_o,
                    uhi_v, ihi_v, uidx_v, iidx_v,
                    ue_v, ie_v, ul_v, ub_v, ib_v, sem):
    wid = lax.axis_index("s") * NC + lax.axis_index("c")
    base = wid * BPW

    pltpu.sync_copy(uhi.at[pl.ds(base, BPW)], uhi_v)
    pltpu.sync_copy(ihi.at[pl.ds(base, BPW)], ihi_v)
    pltpu.sync_copy(uidx.at[pl.ds(base, BPW)], uidx_v)
    pltpu.sync_copy(iidx.at[pl.ds(base, BPW)], iidx_v)

    for j in range(NG):
        s = pl.ds(j * GCH, GCH)
        o = pl.ds(base + j * GCH, GCH)
        copies = [
            pltpu.async_copy(eu.at[uhi_v.at[s]], ue_v, sem),
            pltpu.async_copy(ei.at[ihi_v.at[s]], ie_v, sem),
            pltpu.async_copy(ul.at[uidx_v.at[s]], ul_v, sem),
            pltpu.async_copy(ub.at[uidx_v.at[s]], ub_v, sem),
            pltpu.async_copy(ib.at[iidx_v.at[s]], ib_v, sem),
        ]
        for c in copies:
            c.wait()
        pltpu.sync_copy(ue_v, ue_o.at[o])
        pltpu.sync_copy(ie_v, ie_o.at[o])
        pltpu.sync_copy(ul_v, ul_o.at[o])
        pltpu.sync_copy(ub_v, ub_o.at[o])
        pltpu.sync_copy(ib_v, ib_o.at[o])


def _tc_body(par, ue_s, ie_s, ulo, ilo, r, ul, ub, ib, int_o, dst_o):
    cut = par[0]
    al = par[1]
    be = par[2]
    ga = par[3]
    gl = par[4]
    ulo_v = ulo[...]
    ilo_v = ilo[...]
    ue = jnp.zeros((RB, D), jnp.float32)
    ie = jnp.zeros((RB, D), jnp.float32)
    for k in range(PACK):
        um = (ulo_v == k).astype(jnp.float32)[:, None]
        im = (ilo_v == k).astype(jnp.float32)[:, None]
        ue = ue + um * ue_s[:, k * D:(k + 1) * D]
        ie = ie + im * ie_s[:, k * D:(k + 1) * D]
    bd = jnp.sum(jnp.abs(ie - ue), axis=1)
    rv = r[...]
    m = (rv > 0.0).astype(jnp.float32)
    dt = rv * m + cut
    pv = lax.rsqrt(dt) * m
    bl = jnp.sum(pv, axis=1)
    lam = gl + jnp.clip(ul[...], 0.1, 10.0)
    act = jnp.minimum(bl * lam, bd)
    dist = bd - act
    ival = al * dist + be * dist * dist + ga + ub[...] + ib[...]
    int_o[...] = 1.0 / (1.0 + jnp.exp(-ival))
    dst_o[...] = dist


@jax.jit
def _run(uhi, ihi, uidx, iidx, ulo, ilo, r_interval, eu, ei, ul, ub, ib,
         par):
    mesh = plsc.VectorSubcoreMesh(
        core_axis_name="c", subcore_axis_name="s",
        num_cores=NC, num_subcores=NS)
    gather = pl.kernel(
        _sc_gather_body,
        out_type=(
            jax.ShapeDtypeStruct((B, DW), jnp.float32),
            jax.ShapeDtypeStruct((B, DW), jnp.float32),
            jax.ShapeDtypeStruct((B,), jnp.float32),
            jax.ShapeDtypeStruct((B,), jnp.float32),
            jax.ShapeDtypeStruct((B,), jnp.float32),
        ),
        mesh=mesh,
        scratch_types=[
            pltpu.VMEM((BPW,), jnp.int32),
            pltpu.VMEM((BPW,), jnp.int32),
            pltpu.VMEM((BPW,), jnp.int32),
            pltpu.VMEM((BPW,), jnp.int32),
            pltpu.VMEM((GCH, DW), jnp.float32),
            pltpu.VMEM((GCH, DW), jnp.float32),
            pltpu.VMEM((GCH,), jnp.float32),
            pltpu.VMEM((GCH,), jnp.float32),
            pltpu.VMEM((GCH,), jnp.float32),
            pltpu.SemaphoreType.DMA,
        ],
    )
    ue_g, ie_g, ul_g, ub_g, ib_g = gather(uhi, ihi, uidx, iidx,
                                          eu, ei, ul, ub, ib)

    compute = pl.pallas_call(
        _tc_body,
        grid=(B // RB,),
        in_specs=[
            pl.BlockSpec(memory_space=pltpu.SMEM),
            pl.BlockSpec((RB, DW), lambda i: (i, 0)),
            pl.BlockSpec((RB, DW), lambda i: (i, 0)),
            pl.BlockSpec((RB,), lambda i: (i,)),
            pl.BlockSpec((RB,), lambda i: (i,)),
            pl.BlockSpec((RB, HIST), lambda i: (i, 0)),
            pl.BlockSpec((RB,), lambda i: (i,)),
            pl.BlockSpec((RB,), lambda i: (i,)),
            pl.BlockSpec((RB,), lambda i: (i,)),
        ],
        out_specs=(
            pl.BlockSpec((RB,), lambda i: (i,)),
            pl.BlockSpec((RB,), lambda i: (i,)),
        ),
        out_shape=(
            jax.ShapeDtypeStruct((B,), jnp.float32),
            jax.ShapeDtypeStruct((B,), jnp.float32),
        ),
    )
    return compute(par, ue_g, ie_g, ulo, ilo, r_interval,
                   ul_g, ub_g, ib_g)


def kernel(user_indices, item_indices, r_interval, embedding_user,
           embedding_item, user_lamb, user_bias, item_bias, global_lamb,
           alpha, beta, gamma, cutoff):
    uidx = user_indices.astype(jnp.int32)
    iidx = item_indices.astype(jnp.int32)
    uhi = uidx >> 2
    ihi = iidx >> 2
    ulo = uidx & 3
    ilo = iidx & 3
    cut = jnp.clip(cutoff.astype(jnp.float32), 0.1, 100.0)
    gl = jnp.clip(global_lamb.astype(jnp.float32), 0.01, 10.0)
    par = jnp.stack([cut, alpha.astype(jnp.float32),
                     beta.astype(jnp.float32), gamma.astype(jnp.float32),
                     gl])
    interest, distance = _run(
        uhi, ihi, uidx, iidx, ulo, ilo, r_interval,
        embedding_user.reshape(-1, DW), embedding_item.reshape(-1, DW),
        user_lamb.reshape(-1), user_bias.reshape(-1), item_bias.reshape(-1),
        par)
    return (interest, distance)


# speedup vs baseline: 1.0038x; 1.0038x over previous
import jax
import jax.numpy as jnp
from jax import lax
from jax.experimental import pallas as pl
from jax.experimental.pallas import tpu as pltpu
from jax.experimental.pallas import tpu_sc as plsc

NC = 2
NS = 16
NW = NC * NS

B = 16384
D = 32
PACK = 128 // D
HIST = 50
BPW = B // NW
GCH = 128
NG = BPW // GCH
RCH = 64
NR = BPW // RCH

RB = 2048
GRID = B // RB


def _sc_gather_body(uidx, iidx, uhi, ihi, ul, ub, ib, eu, ei,
                    ul_o, ub_o, ib_o, ue_o, ie_o,
                    uidx_v, iidx_v, uhi_v, ihi_v,
                    ul_v, ub_v, ib_v, ue_v, ie_v, sem):
    wid = lax.axis_index("s") * NC + lax.axis_index("c")
    base = wid * BPW

    pltpu.sync_copy(uidx.at[pl.ds(base, BPW)], uidx_v)
    pltpu.sync_copy(iidx.at[pl.ds(base, BPW)], iidx_v)
    pltpu.sync_copy(uhi.at[pl.ds(base, BPW)], uhi_v)
    pltpu.sync_copy(ihi.at[pl.ds(base, BPW)], ihi_v)

    copies = []
    for j in range(NG):
        s = pl.ds(j * GCH, GCH)
        copies.append(pltpu.async_copy(ul.at[uidx_v.at[s]], ul_v.at[s], sem))
        copies.append(pltpu.async_copy(ub.at[uidx_v.at[s]], ub_v.at[s], sem))
        copies.append(pltpu.async_copy(ib.at[iidx_v.at[s]], ib_v.at[s], sem))
    for c in copies:
        c.wait()

    out = pl.ds(base, BPW)
    pltpu.sync_copy(ul_v, ul_o.at[out])
    pltpu.sync_copy(ub_v, ub_o.at[out])
    pltpu.sync_copy(ib_v, ib_o.at[out])

    for k in range(NR):
        s = pl.ds(k * RCH, RCH)
        cu = pltpu.async_copy(eu.at[uhi_v.at[s]], ue_v, sem)
        ci = pltpu.async_copy(ei.at[ihi_v.at[s]], ie_v, sem)
        cu.wait()
        ci.wait()
        o = pl.ds(base + k * RCH, RCH)
        pltpu.sync_copy(ue_v, ue_o.at[o])
        pltpu.sync_copy(ie_v, ie_o.at[o])


def _tc_row_body(par, ue, ie, ulo, ilo, r, bd_o, bl_o):
    ulov = ulo[...]
    ilov = ilo[...]
    uev = ue[...]
    iev = ie[...]
    su = jnp.zeros((RB, D), jnp.float32)
    si = jnp.zeros((RB, D), jnp.float32)
    for j in range(PACK):
        mu = (ulov == j).astype(jnp.float32)[:, None]
        mi = (ilov == j).astype(jnp.float32)[:, None]
        su = su + mu * uev[:, j * D:(j + 1) * D]
        si = si + mi * iev[:, j * D:(j + 1) * D]
    bd_o[...] = jnp.sum(jnp.abs(si - su), axis=1)

    rv = r[...]
    m = (rv > 0.0).astype(jnp.float32)
    dt = rv * m + par[0]
    pv = lax.rsqrt(dt) * m
    bl_o[...] = jnp.sum(pv, axis=1)


def _tc_combine_body(par, bd, bl, ul, ub, ib, int_o, dst_o):
    al = par[1]
    be = par[2]
    ga = par[3]
    gl = par[4]
    bdv = bd[...]
    lam = gl + jnp.clip(ul[...], 0.1, 10.0)
    act = jnp.minimum(bl[...] * lam, bdv)
    dist = bdv - act
    ival = al * dist + be * dist * dist + ga + ub[...] + ib[...]
    int_o[...] = 1.0 / (1.0 + jnp.exp(-ival))
    dst_o[...] = dist


@jax.jit
def _run(uidx, iidx, uhi, ihi, ulo, ilo, r_interval,
         eu128, ei128, ul, ub, ib, par):
    mesh = plsc.VectorSubcoreMesh(
        core_axis_name="c", subcore_axis_name="s",
        num_cores=NC, num_subcores=NS)
    sc_gather = pl.kernel(
        _sc_gather_body,
        out_type=(
            jax.ShapeDtypeStruct((B,), jnp.float32),
            jax.ShapeDtypeStruct((B,), jnp.float32),
            jax.ShapeDtypeStruct((B,), jnp.float32),
            jax.ShapeDtypeStruct((B, 128), jnp.float32),
            jax.ShapeDtypeStruct((B, 128), jnp.float32),
        ),
        mesh=mesh,
        scratch_types=[
            pltpu.VMEM((BPW,), jnp.int32),
            pltpu.VMEM((BPW,), jnp.int32),
            pltpu.VMEM((BPW,), jnp.int32),
            pltpu.VMEM((BPW,), jnp.int32),
            pltpu.VMEM((BPW,), jnp.float32),
            pltpu.VMEM((BPW,), jnp.float32),
            pltpu.VMEM((BPW,), jnp.float32),
            pltpu.VMEM((RCH, 128), jnp.float32),
            pltpu.VMEM((RCH, 128), jnp.float32),
            pltpu.SemaphoreType.DMA,
        ],
    )
    ul_g, ub_g, ib_g, ue_g, ie_g = sc_gather(
        uidx, iidx, uhi, ihi, ul, ub, ib, eu128, ei128)

    row_math = pl.pallas_call(
        _tc_row_body,
        grid=(GRID,),
        in_specs=[
            pl.BlockSpec(memory_space=pltpu.SMEM),
            pl.BlockSpec((RB, 128), lambda i: (i, 0)),
            pl.BlockSpec((RB, 128), lambda i: (i, 0)),
            pl.BlockSpec((RB,), lambda i: (i,)),
            pl.BlockSpec((RB,), lambda i: (i,)),
            pl.BlockSpec((RB, HIST), lambda i: (i, 0)),
        ],
        out_specs=(
            pl.BlockSpec((RB,), lambda i: (i,)),
            pl.BlockSpec((RB,), lambda i: (i,)),
        ),
        out_shape=(
            jax.ShapeDtypeStruct((B,), jnp.float32),
            jax.ShapeDtypeStruct((B,), jnp.float32),
        ),
    )
    bd, bl = row_math(par, ue_g, ie_g, ulo, ilo, r_interval)

    combine = pl.pallas_call(
        _tc_combine_body,
        grid=(1,),
        in_specs=[
            pl.BlockSpec(memory_space=pltpu.SMEM),
            pl.BlockSpec((B,), lambda i: (i,)),
            pl.BlockSpec((B,), lambda i: (i,)),
            pl.BlockSpec((B,), lambda i: (i,)),
            pl.BlockSpec((B,), lambda i: (i,)),
            pl.BlockSpec((B,), lambda i: (i,)),
        ],
        out_specs=(
            pl.BlockSpec((B,), lambda i: (i,)),
            pl.BlockSpec((B,), lambda i: (i,)),
        ),
        out_shape=(
            jax.ShapeDtypeStruct((B,), jnp.float32),
            jax.ShapeDtypeStruct((B,), jnp.float32),
        ),
    )
    return combine(par, bd, bl, ul_g, ub_g, ib_g)


def kernel(user_indices, item_indices, r_interval, embedding_user,
           embedding_item, user_lamb, user_bias, item_bias, global_lamb,
           alpha, beta, gamma, cutoff):
    uidx = user_indices.astype(jnp.int32)
    iidx = item_indices.astype(jnp.int32)
    uhi = uidx // PACK
    ihi = iidx // PACK
    ulo = uidx % PACK
    ilo = iidx % PACK
    cut = jnp.clip(cutoff.astype(jnp.float32), 0.1, 100.0)
    gl = jnp.clip(global_lamb.astype(jnp.float32), 0.01, 10.0)
    par = jnp.stack([cut, alpha.astype(jnp.float32),
                     beta.astype(jnp.float32), gamma.astype(jnp.float32),
                     gl])
    interest, distance = _run(
        uidx, iidx, uhi, ihi, ulo, ilo, r_interval,
        embedding_user.reshape(-1, 128), embedding_item.reshape(-1, 128),
        user_lamb.reshape(-1), user_bias.reshape(-1), item_bias.reshape(-1),
        par)
    return (interest, distance)
